# baseline (speedup 1.0000x reference)
import functools

import jax
import jax.numpy as jnp
from jax import lax
from jax.experimental import pallas as pl
from jax.experimental.pallas import tpu as pltpu
from jax.experimental.pallas import tpu_sc as plsc

_N = 10000
_E = 320000
_D = 128
_H = 32
_OUT = 64
_G = 16

_NC = 2
_NS = 16
_NW = _NC * _NS
_EP = _E // _NW
_WCH = 125
_NWIN = _EP // _WCH
_NP = 10112
_RP = _NP // _NS
_DW = 8

_mesh = plsc.VectorSubcoreMesh(core_axis_name="c", subcore_axis_name="s")


@functools.partial(
    pl.kernel,
    out_type=jax.ShapeDtypeStruct((_NC, _NP, _DW), jnp.float32),
    mesh=_mesh,
    compiler_params=pltpu.CompilerParams(use_tc_tiling_on_sc=False),
    scratch_types=[
        pltpu.VMEM((_NWIN, _WCH), jnp.int32),
        pltpu.VMEM((_WCH, _DW), jnp.float32),
        pltpu.VMEM_SHARED((_NP, _DW), jnp.float32),
        pltpu.SemaphoreType.DMA,
        pltpu.SemaphoreType.DMA,
        pltpu.SemaphoreType.DMA,
        pltpu.SemaphoreType.DMA,
    ],
)
def _deg_sc(dst_hbm, ones_hbm, zeros_hbm, out_hbm, idx_v, ones_v, acc_sh,
            ss0, ss1, ss2, ss3):
    c = lax.axis_index("c")
    s = lax.axis_index("s")
    wid = s * _NC + c
    base = pl.multiple_of(s * _RP, 8)
    pltpu.sync_copy(dst_hbm.at[wid], idx_v)
    pltpu.sync_copy(ones_hbm, ones_v)
    pltpu.sync_copy(zeros_hbm.at[pl.ds(base, _RP)], acc_sh.at[pl.ds(base, _RP)])
    plsc.subcore_barrier()

    ssems = (ss0, ss1, ss2, ss3)

    def body(i, carry):
        jbase = 4 * i
        for k in range(4):
            pltpu.async_copy(ones_v, acc_sh.at[idx_v.at[jbase + k]], ssems[k],
                             add=True)
        for k in range(4):
            pltpu.make_async_copy(ones_v, acc_sh.at[idx_v.at[jbase + k]],
                                  ssems[k]).wait()
        return carry

    lax.fori_loop(0, _NWIN // 4, body, 0)
    plsc.subcore_barrier()
    pltpu.sync_copy(acc_sh.at[pl.ds(base, _RP)], out_hbm.at[c, pl.ds(base, _RP)])


@functools.partial(
    pl.kernel,
    out_type=jax.ShapeDtypeStruct((_NC, _NP, _H), jnp.float32),
    mesh=_mesh,
    compiler_params=pltpu.CompilerParams(use_tc_tiling_on_sc=False),
    scratch_types=[
        pltpu.VMEM((_NWIN, _WCH), jnp.int32),
        pltpu.VMEM((_NWIN, _WCH), jnp.int32),
        pltpu.VMEM((_WCH, _H), jnp.float32),
        pltpu.VMEM((_WCH, _H), jnp.float32),
        pltpu.VMEM((_WCH, _H), jnp.float32),
        pltpu.VMEM((_WCH, _H), jnp.float32),
        pltpu.VMEM_SHARED((_NP, _H), jnp.float32),
        pltpu.VMEM_SHARED((_NP, _H), jnp.float32),
        pltpu.SemaphoreType.DMA,
        pltpu.SemaphoreType.DMA,
        pltpu.SemaphoreType.DMA,
        pltpu.SemaphoreType.DMA,
        pltpu.SemaphoreType.DMA,
        pltpu.SemaphoreType.DMA,
        pltpu.SemaphoreType.DMA,
        pltpu.SemaphoreType.DMA,
    ],
)
def _agg_sc(p_hbm, src_hbm, dst_hbm, zeros_hbm, out_hbm,
            si_v, di_v, r0, r1, r2, r3, acc_sh, p_sh,
            gs0, gs1, gs2, gs3, ss0, ss1, ss2, ss3):
    c = lax.axis_index("c")
    s = lax.axis_index("s")
    wid = s * _NC + c
    base = pl.multiple_of(s * _RP, 8)
    pltpu.sync_copy(src_hbm.at[wid], si_v)
    pltpu.sync_copy(dst_hbm.at[wid], di_v)
    pltpu.sync_copy(p_hbm.at[pl.ds(base, _RP)], p_sh.at[pl.ds(base, _RP)])
    pltpu.sync_copy(zeros_hbm.at[pl.ds(base, _RP)], acc_sh.at[pl.ds(base, _RP)])
    plsc.subcore_barrier()

    bufs = (r0, r1, r2, r3)
    gsems = (gs0, gs1, gs2, gs3)
    ssems = (ss0, ss1, ss2, ss3)
    for k in range(4):
        pltpu.async_copy(p_sh.at[si_v.at[k]], bufs[k], gsems[k])

    def body(i, carry):
        jbase = 4 * i
        for k in range(4):
            j = jbase + k
            pltpu.make_async_copy(p_sh.at[si_v.at[j]], bufs[k], gsems[k]).wait()
            pltpu.async_copy(bufs[k], acc_sh.at[di_v.at[j]], ssems[k], add=True)
        for k in range(4):
            j = jbase + k
            pltpu.make_async_copy(bufs[k], acc_sh.at[di_v.at[j]], ssems[k]).wait()

            @pl.when(j + 4 < _NWIN)
            def _():
                pltpu.async_copy(p_sh.at[si_v.at[j + 4]], bufs[k], gsems[k])
        return carry

    lax.fori_loop(0, _NWIN // 4, body, 0)
    plsc.subcore_barrier()
    pltpu.sync_copy(acc_sh.at[pl.ds(base, _RP)], out_hbm.at[c, pl.ds(base, _RP)])


def _dinv_from(degp):
    deg = degp[0, :_N, 0] + degp[1, :_N, 0] + 1.0
    return lax.rsqrt(jnp.maximum(deg, 1.0))


def _dense1_body(degp_ref, x_ref, w1_ref, out_ref):
    dinv = _dinv_from(degp_ref[...])
    h = jnp.dot(x_ref[...], w1_ref[...], preferred_element_type=jnp.float32)
    out_ref[pl.ds(0, _N), :] = h * dinv[:, None]


def _dense_mid_body(agg_ref, p_ref, degp_ref, b_ref, w_ref, out_ref):
    dinv = _dinv_from(degp_ref[...])
    a = agg_ref[...]
    p = p_ref[...][: _N]
    z = jnp.maximum((a[0, :_N] + a[1, :_N] + p) * dinv[:, None] + b_ref[...][None, :], 0.0)
    h = jnp.dot(z, w_ref[...], preferred_element_type=jnp.float32)
    out_ref[pl.ds(0, _N), :] = h * dinv[:, None]


def _dense_out_body(agg_ref, p_ref, degp_ref, b_ref, batch_ref, wo_ref, bo_ref, out_ref):
    dinv = _dinv_from(degp_ref[...])
    a = agg_ref[...]
    p = p_ref[...][: _N]
    z = jnp.maximum((a[0, :_N] + a[1, :_N] + p) * dinv[:, None] + b_ref[...][None, :], 0.0)
    oh = (batch_ref[...][:, None] == lax.broadcasted_iota(jnp.int32, (1, _G), 1)
          ).astype(jnp.float32)
    counts = jnp.sum(oh, axis=0)
    summed = lax.dot_general(oh, z, (((0,), (0,)), ((), ())),
                             preferred_element_type=jnp.float32)
    pooled = summed / jnp.maximum(counts, 1.0)[:, None]
    out_ref[...] = (jnp.dot(pooled, wo_ref[...], preferred_element_type=jnp.float32)
                    + bo_ref[...][None, :])


_dense1 = pl.pallas_call(
    _dense1_body, out_shape=jax.ShapeDtypeStruct((_NP, _H), jnp.float32))
_dense_mid = pl.pallas_call(
    _dense_mid_body, out_shape=jax.ShapeDtypeStruct((_NP, _H), jnp.float32))
_dense_out = pl.pallas_call(
    _dense_out_body, out_shape=jax.ShapeDtypeStruct((_G, _OUT), jnp.float32))


def kernel(x, edge_index, batch, W1, b1, W2, b2, Wo, bo):
    src = edge_index[0].reshape(_NW, _NWIN, _WCH)
    dst = edge_index[1].reshape(_NW, _NWIN, _WCH)
    zeros_h = jnp.zeros((_NP, _H), jnp.float32)
    zeros_d = jnp.zeros((_NP, _DW), jnp.float32)
    ones_w = jnp.ones((_WCH, _DW), jnp.float32)

    degp = _deg_sc(dst, ones_w, zeros_d)
    p1 = _dense1(degp, x, W1)
    agg1 = _agg_sc(p1, src, dst, zeros_h)
    p2 = _dense_mid(agg1, p1, degp, b1, W2)
    agg2 = _agg_sc(p2, src, dst, zeros_h)
    return _dense_out(agg2, p2, degp, b2, batch, Wo, bo)


# speedup vs baseline: 1.0423x; 1.0423x over previous
import functools

import jax
import jax.numpy as jnp
from jax import lax
from jax.experimental import pallas as pl
from jax.experimental.pallas import tpu as pltpu
from jax.experimental.pallas import tpu_sc as plsc

_N = 10000
_E = 320000
_D = 128
_H = 32
_OUT = 64
_G = 16

_NC = 2
_NS = 16
_NW = _NC * _NS
_EP = _E // _NW
_WCH = 125
_NWIN = _EP // _WCH
_NP = 10112
_RP = _NP // _NS
_DW = 8

_mesh = plsc.VectorSubcoreMesh(core_axis_name="c", subcore_axis_name="s")


@functools.partial(
    pl.kernel,
    out_type=jax.ShapeDtypeStruct((_NC, _NP, _DW), jnp.float32),
    mesh=_mesh,
    compiler_params=pltpu.CompilerParams(use_tc_tiling_on_sc=False),
    scratch_types=[
        pltpu.VMEM((_NWIN, _WCH), jnp.int32),
        pltpu.VMEM((_WCH, _DW), jnp.float32),
        pltpu.VMEM_SHARED((_NP, _DW), jnp.float32),
    ],
)
def _deg_sc(dst_hbm, ones_hbm, zeros_hbm, out_hbm, idx_v, ones_v, acc_sh):
    c = lax.axis_index("c")
    s = lax.axis_index("s")
    wid = s * _NC + c
    base = pl.multiple_of(s * _RP, 8)
    pltpu.sync_copy(dst_hbm.at[wid], idx_v)
    pltpu.sync_copy(ones_hbm, ones_v)
    pltpu.sync_copy(zeros_hbm.at[pl.ds(base, _RP)], acc_sh.at[pl.ds(base, _RP)])
    plsc.subcore_barrier()

    def body(j, carry):
        pltpu.sync_copy(ones_v, acc_sh.at[idx_v.at[j]], add=True)
        return carry

    lax.fori_loop(0, _NWIN, body, 0)
    plsc.subcore_barrier()
    pltpu.sync_copy(acc_sh.at[pl.ds(base, _RP)], out_hbm.at[c, pl.ds(base, _RP)])


@functools.partial(
    pl.kernel,
    out_type=jax.ShapeDtypeStruct((_NC, _NP, _H), jnp.float32),
    mesh=_mesh,
    compiler_params=pltpu.CompilerParams(use_tc_tiling_on_sc=False),
    scratch_types=[
        pltpu.VMEM((_NWIN, _WCH), jnp.int32),
        pltpu.VMEM((_NWIN, _WCH), jnp.int32),
        pltpu.VMEM((_WCH, _H), jnp.float32),
        pltpu.VMEM((_WCH, _H), jnp.float32),
        pltpu.VMEM_SHARED((_NP, _H), jnp.float32),
        pltpu.VMEM_SHARED((_NP, _H), jnp.float32),
        pltpu.SemaphoreType.DMA,
        pltpu.SemaphoreType.DMA,
    ],
)
def _agg_sc(p_hbm, src_hbm, dst_hbm, zeros_hbm, out_hbm,
            si_v, di_v, r0, r1, acc_sh, p_sh, sem0, sem1):
    c = lax.axis_index("c")
    s = lax.axis_index("s")
    wid = s * _NC + c
    base = pl.multiple_of(s * _RP, 8)
    pltpu.sync_copy(src_hbm.at[wid], si_v)
    pltpu.sync_copy(dst_hbm.at[wid], di_v)
    pltpu.sync_copy(p_hbm.at[pl.ds(base, _RP)], p_sh.at[pl.ds(base, _RP)])
    pltpu.sync_copy(zeros_hbm.at[pl.ds(base, _RP)], acc_sh.at[pl.ds(base, _RP)])
    plsc.subcore_barrier()

    pltpu.async_copy(p_sh.at[si_v.at[0]], r0, sem0)

    def body(i, carry):
        j0 = 2 * i
        j1 = j0 + 1
        pltpu.async_copy(p_sh.at[si_v.at[j1]], r1, sem1)
        pltpu.make_async_copy(p_sh.at[si_v.at[j0]], r0, sem0).wait()
        pltpu.sync_copy(r0, acc_sh.at[di_v.at[j0]], add=True)

        @pl.when(j0 + 2 < _NWIN)
        def _():
            pltpu.async_copy(p_sh.at[si_v.at[j0 + 2]], r0, sem0)

        pltpu.make_async_copy(p_sh.at[si_v.at[j1]], r1, sem1).wait()
        pltpu.sync_copy(r1, acc_sh.at[di_v.at[j1]], add=True)
        return carry

    lax.fori_loop(0, _NWIN // 2, body, 0)
    plsc.subcore_barrier()
    pltpu.sync_copy(acc_sh.at[pl.ds(base, _RP)], out_hbm.at[c, pl.ds(base, _RP)])


def _dinv_from(degp):
    deg = degp[0, :_N, 0] + degp[1, :_N, 0] + 1.0
    return lax.rsqrt(jnp.maximum(deg, 1.0))


def _dense1_body(degp_ref, x_ref, w1_ref, out_ref):
    dinv = _dinv_from(degp_ref[...])
    h = jnp.dot(x_ref[...], w1_ref[...], preferred_element_type=jnp.float32)
    out_ref[pl.ds(0, _N), :] = h * dinv[:, None]


def _dense_mid_body(agg_ref, p_ref, degp_ref, b_ref, w_ref, out_ref):
    dinv = _dinv_from(degp_ref[...])
    a = agg_ref[...]
    p = p_ref[...][: _N]
    z = jnp.maximum((a[0, :_N] + a[1, :_N] + p) * dinv[:, None] + b_ref[...][None, :], 0.0)
    h = jnp.dot(z, w_ref[...], preferred_element_type=jnp.float32)
    out_ref[pl.ds(0, _N), :] = h * dinv[:, None]


def _dense_out_body(agg_ref, p_ref, degp_ref, b_ref, batch_ref, wo_ref, bo_ref, out_ref):
    dinv = _dinv_from(degp_ref[...])
    a = agg_ref[...]
    p = p_ref[...][: _N]
    z = jnp.maximum((a[0, :_N] + a[1, :_N] + p) * dinv[:, None] + b_ref[...][None, :], 0.0)
    oh = (batch_ref[...][:, None] == lax.broadcasted_iota(jnp.int32, (1, _G), 1)
          ).astype(jnp.float32)
    counts = jnp.sum(oh, axis=0)
    summed = lax.dot_general(oh, z, (((0,), (0,)), ((), ())),
                             preferred_element_type=jnp.float32)
    pooled = summed / jnp.maximum(counts, 1.0)[:, None]
    out_ref[...] = (jnp.dot(pooled, wo_ref[...], preferred_element_type=jnp.float32)
                    + bo_ref[...][None, :])


_dense1 = pl.pallas_call(
    _dense1_body, out_shape=jax.ShapeDtypeStruct((_NP, _H), jnp.float32))
_dense_mid = pl.pallas_call(
    _dense_mid_body, out_shape=jax.ShapeDtypeStruct((_NP, _H), jnp.float32))
_dense_out = pl.pallas_call(
    _dense_out_body, out_shape=jax.ShapeDtypeStruct((_G, _OUT), jnp.float32))


def kernel(x, edge_index, batch, W1, b1, W2, b2, Wo, bo):
    src = edge_index[0].reshape(_NW, _NWIN, _WCH)
    dst = edge_index[1].reshape(_NW, _NWIN, _WCH)
    zeros_h = jnp.zeros((_NP, _H), jnp.float32)
    zeros_d = jnp.zeros((_NP, _DW), jnp.float32)
    ones_w = jnp.ones((_WCH, _DW), jnp.float32)

    degp = _deg_sc(dst, ones_w, zeros_d)
    p1 = _dense1(degp, x, W1)
    agg1 = _agg_sc(p1, src, dst, zeros_h)
    p2 = _dense_mid(agg1, p1, degp, b1, W2)
    agg2 = _agg_sc(p2, src, dst, zeros_h)
    return _dense_out(agg2, p2, degp, b2, batch, Wo, bo)


# speedup vs baseline: 1.0488x; 1.0062x over previous
import functools

import jax
import jax.numpy as jnp
from jax import lax
from jax.experimental import pallas as pl
from jax.experimental.pallas import tpu as pltpu
from jax.experimental.pallas import tpu_sc as plsc

_N = 10000
_E = 320000
_D = 128
_H = 32
_OUT = 64
_G = 16

_NC = 2
_NS = 16
_NW = _NC * _NS
_EP = _E // _NW
_WCH = 125
_NWIN = _EP // _WCH
_NP = 10112
_RP = _NP // _NS
_DW = 8

_mesh = plsc.VectorSubcoreMesh(core_axis_name="c", subcore_axis_name="s")


@functools.partial(
    pl.kernel,
    out_type=jax.ShapeDtypeStruct((_NC, _NP, _DW), jnp.float32),
    mesh=_mesh,
    compiler_params=pltpu.CompilerParams(use_tc_tiling_on_sc=False),
    scratch_types=[
        pltpu.VMEM((_NWIN, _WCH), jnp.int32),
        pltpu.VMEM((_WCH, _DW), jnp.float32),
        pltpu.VMEM_SHARED((_NP, _DW), jnp.float32),
    ],
)
def _deg_sc(dst_hbm, ones_hbm, zeros_hbm, out_hbm, idx_v, ones_v, acc_sh):
    c = lax.axis_index("c")
    s = lax.axis_index("s")
    wid = s * _NC + c
    base = pl.multiple_of(s * _RP, 8)
    pltpu.sync_copy(dst_hbm.at[wid], idx_v)
    pltpu.sync_copy(ones_hbm, ones_v)
    pltpu.sync_copy(zeros_hbm.at[pl.ds(base, _RP)], acc_sh.at[pl.ds(base, _RP)])
    plsc.subcore_barrier()

    def body(j, carry):
        pltpu.sync_copy(ones_v, acc_sh.at[idx_v.at[j]], add=True)
        return carry

    lax.fori_loop(0, _NWIN, body, 0)
    plsc.subcore_barrier()
    pltpu.sync_copy(acc_sh.at[pl.ds(base, _RP)], out_hbm.at[c, pl.ds(base, _RP)])


@functools.partial(
    pl.kernel,
    out_type=jax.ShapeDtypeStruct((_NC, _NP, _H), jnp.float32),
    mesh=_mesh,
    compiler_params=pltpu.CompilerParams(use_tc_tiling_on_sc=False),
    scratch_types=[
        pltpu.VMEM((_NWIN, _WCH), jnp.int32),
        pltpu.VMEM((_NWIN, _WCH), jnp.int32),
        pltpu.VMEM((_WCH, _H), jnp.float32),
        pltpu.VMEM((_WCH, _H), jnp.float32),
        pltpu.VMEM_SHARED((_NP, _H), jnp.float32),
        pltpu.VMEM_SHARED((_NP, _H), jnp.float32),
        pltpu.SemaphoreType.DMA,
        pltpu.SemaphoreType.DMA,
    ],
)
def _agg_sc(p_hbm, src_hbm, dst_hbm, zeros_hbm, out_hbm,
            si_v, di_v, r0, r1, acc_sh, p_sh, sem0, sem1):
    c = lax.axis_index("c")
    s = lax.axis_index("s")
    wid = s * _NC + c
    base = pl.multiple_of(s * _RP, 8)
    pltpu.sync_copy(src_hbm.at[wid], si_v)
    pltpu.sync_copy(dst_hbm.at[wid], di_v)
    pltpu.sync_copy(p_hbm.at[pl.ds(base, _RP)], p_sh.at[pl.ds(base, _RP)])
    pltpu.sync_copy(zeros_hbm.at[pl.ds(base, _RP)], acc_sh.at[pl.ds(base, _RP)])
    plsc.subcore_barrier()

    pltpu.async_copy(p_sh.at[si_v.at[0]], r0, sem0)

    def body(i, carry):
        j0 = 2 * i
        j1 = j0 + 1
        pltpu.async_copy(p_sh.at[si_v.at[j1]], r1, sem1)
        pltpu.make_async_copy(p_sh.at[si_v.at[j0]], r0, sem0).wait()
        pltpu.sync_copy(r0, acc_sh.at[di_v.at[j0]], add=True)

        @pl.when(j0 + 2 < _NWIN)
        def _():
            pltpu.async_copy(p_sh.at[si_v.at[j0 + 2]], r0, sem0)

        pltpu.make_async_copy(p_sh.at[si_v.at[j1]], r1, sem1).wait()
        pltpu.sync_copy(r1, acc_sh.at[di_v.at[j1]], add=True)
        return carry

    lax.fori_loop(0, _NWIN // 2, body, 0)
    plsc.subcore_barrier()
    pltpu.sync_copy(acc_sh.at[pl.ds(base, _RP)], out_hbm.at[c, pl.ds(base, _RP)])


def _dinv_from(degp):
    deg = degp[0, :_N, 0] + degp[1, :_N, 0] + 1.0
    return lax.rsqrt(jnp.maximum(deg, 1.0))


_BLK = 2000


def _dense1_body(degp_ref, x_ref, w1_ref, out_ref):
    deg = degp_ref[0, :, 0] + degp_ref[1, :, 0] + 1.0
    dinv = lax.rsqrt(jnp.maximum(deg, 1.0))
    h = jnp.dot(x_ref[...], w1_ref[...], preferred_element_type=jnp.float32)
    out_ref[...] = h * dinv[:, None]


def _dense_mid_body(agg_ref, p_ref, degp_ref, b_ref, w_ref, out_ref):
    deg = degp_ref[0, :, 0] + degp_ref[1, :, 0] + 1.0
    dinv = lax.rsqrt(jnp.maximum(deg, 1.0))
    a = agg_ref[...]
    z = jnp.maximum((a[0] + a[1] + p_ref[...]) * dinv[:, None] + b_ref[...][None, :], 0.0)
    h = jnp.dot(z, w_ref[...], preferred_element_type=jnp.float32)
    out_ref[...] = h * dinv[:, None]


def _dense_out_body(agg_ref, p_ref, degp_ref, b_ref, batch_ref, wo_ref, bo_ref, out_ref):
    dinv = _dinv_from(degp_ref[...])
    a = agg_ref[...]
    p = p_ref[...][: _N]
    z = jnp.maximum((a[0, :_N] + a[1, :_N] + p) * dinv[:, None] + b_ref[...][None, :], 0.0)
    oh = (batch_ref[...][:, None] == lax.broadcasted_iota(jnp.int32, (1, _G), 1)
          ).astype(jnp.float32)
    counts = jnp.sum(oh, axis=0)
    summed = lax.dot_general(oh, z, (((0,), (0,)), ((), ())),
                             preferred_element_type=jnp.float32)
    pooled = summed / jnp.maximum(counts, 1.0)[:, None]
    out_ref[...] = (jnp.dot(pooled, wo_ref[...], preferred_element_type=jnp.float32)
                    + bo_ref[...][None, :])


_dense1 = pl.pallas_call(
    _dense1_body,
    grid=(_N // _BLK,),
    in_specs=[
        pl.BlockSpec((2, _BLK, _DW), lambda i: (0, i, 0)),
        pl.BlockSpec((_BLK, _D), lambda i: (i, 0)),
        pl.BlockSpec((_D, _H), lambda i: (0, 0)),
    ],
    out_specs=pl.BlockSpec((_BLK, _H), lambda i: (i, 0)),
    out_shape=jax.ShapeDtypeStruct((_NP, _H), jnp.float32))
_dense_mid = pl.pallas_call(
    _dense_mid_body,
    grid=(_N // _BLK,),
    in_specs=[
        pl.BlockSpec((2, _BLK, _H), lambda i: (0, i, 0)),
        pl.BlockSpec((_BLK, _H), lambda i: (i, 0)),
        pl.BlockSpec((2, _BLK, _DW), lambda i: (0, i, 0)),
        pl.BlockSpec((_H,), lambda i: (0,)),
        pl.BlockSpec((_H, _H), lambda i: (0, 0)),
    ],
    out_specs=pl.BlockSpec((_BLK, _H), lambda i: (i, 0)),
    out_shape=jax.ShapeDtypeStruct((_NP, _H), jnp.float32))
_dense_out = pl.pallas_call(
    _dense_out_body, out_shape=jax.ShapeDtypeStruct((_G, _OUT), jnp.float32))


def kernel(x, edge_index, batch, W1, b1, W2, b2, Wo, bo):
    src = edge_index[0].reshape(_NW, _NWIN, _WCH)
    dst = edge_index[1].reshape(_NW, _NWIN, _WCH)
    zeros_h = jnp.zeros((_NP, _H), jnp.float32)
    zeros_d = jnp.zeros((_NP, _DW), jnp.float32)
    ones_w = jnp.ones((_WCH, _DW), jnp.float32)

    degp = _deg_sc(dst, ones_w, zeros_d)
    p1 = _dense1(degp, x, W1)
    agg1 = _agg_sc(p1, src, dst, zeros_h)
    p2 = _dense_mid(agg1, p1, degp, b1, W2)
    agg2 = _agg_sc(p2, src, dst, zeros_h)
    return _dense_out(agg2, p2, degp, b2, batch, Wo, bo)
